# baseline (speedup 1.0000x reference)
import numpy as np

import jax
import jax.numpy as jnp
from jax.experimental import pallas as pl
from jax.experimental.pallas import tpu as pltpu

_L = 2048
_NF = 1024
_K = 5


def _build_tables():
    n1 = np.arange(1024, dtype=np.float64)[:, None]
    j1 = np.arange(512, dtype=np.float64)[None, :]
    ang_o = (2.0 * np.pi / 2048.0) * (2.0 * j1 + 1.0) * n1
    t_odd = np.concatenate([np.cos(ang_o), np.sin(ang_o)], axis=1)

    n2 = np.arange(512, dtype=np.float64)[:, None]
    j2 = np.arange(256, dtype=np.float64)[None, :]
    ang_e2 = (2.0 * np.pi / 1024.0) * (2.0 * j2 + 1.0) * n2
    t_e2 = np.concatenate([np.cos(ang_e2), np.sin(ang_e2)], axis=1)
    ang_e4 = (2.0 * np.pi / 512.0) * (j2 + 1.0) * n2
    t_e4 = np.concatenate([np.cos(ang_e4), np.sin(ang_e4)], axis=1)

    return (t_odd.astype(np.float32).reshape(2, 512, 1024),
            t_e2.astype(np.float32),
            t_e4.astype(np.float32))


_T_ODD, _T_E2, _T_E4 = _build_tables()


def _hdot(a, b):
    return jax.lax.dot_general(
        a, b, (((0,), (0,)), ((), ())),
        preferred_element_type=jnp.float32,
        precision=jax.lax.Precision.HIGHEST)


def _bdot(a, b):
    return jax.lax.dot_general(
        a, b, (((1,), (0,)), ((), ())),
        preferred_element_type=jnp.float32)


def _freq_attn_kernel(to_ref, te2_ref, te4_ref, tob_ref, te2b_ref, te4b_ref,
                      z_ref, o_ref, buf_ref):
    s = pl.program_id(0)

    zb = z_ref[0]
    q0, q1, q2, q3 = zb[0], zb[1], zb[2], zb[3]
    c0 = q0 - q2
    c1 = q1 - q3
    a0 = q0 + q2
    a1 = q1 + q3
    ap = a0 + a1
    cp = a0 - a1

    d_odd = _hdot(to_ref[0], c0) + _hdot(to_ref[1], c1)
    d_e2 = _hdot(te2_ref[...], cp)
    d_e4 = _hdot(te4_ref[...], ap)
    buf_ref[s % 2] = jnp.concatenate([d_odd, d_e2, d_e4], axis=0)

    X = buf_ref[(s + 1) % 2]
    odd = X[:1024]
    e2 = X[1024:1536]
    e4 = X[1536:]

    Re = jnp.concatenate([odd[:512], e2[:256], e4[:256]], axis=0)
    Im = jnp.concatenate([odd[512:], e2[256:], e4[256:]], axis=0)
    amp = jnp.sqrt(Re * Re + Im * Im)
    r = jax.lax.broadcasted_iota(jnp.int32, amp.shape, 0)
    fidx = jnp.where(r < 512, 2 * r + 1,
                     jnp.where(r < 768, 4 * r - 2046, 4 * r - 3068))

    def body(_, w):
        m = jnp.max(w, axis=0, keepdims=True)
        idx = jnp.min(jnp.where(w == m, fidx, 2 * _NF), axis=0, keepdims=True)
        return jnp.where(fidx == idx, -1.0, w)

    work = jax.lax.fori_loop(0, _K, body, amp)
    sel = work < 0.0

    spec_o = jnp.where(jnp.concatenate([sel[:512], sel[:512]], axis=0),
                       odd, 0.0).astype(jnp.bfloat16)
    spec_e2 = jnp.where(jnp.concatenate([sel[512:768], sel[512:768]], axis=0),
                        e2, 0.0).astype(jnp.bfloat16)
    spec_e4 = jnp.where(jnp.concatenate([sel[768:], sel[768:]], axis=0),
                        e4, 0.0).astype(jnp.bfloat16)

    so = _bdot(tob_ref[...], spec_o)
    se2 = _bdot(te2b_ref[...], spec_e2)
    se4 = _bdot(te4b_ref[...], spec_e4)
    se = jnp.concatenate([se4 + se2, se4 - se2], axis=0)
    o_ref[0] = jnp.concatenate([se + so, se - so], axis=0)


def kernel(Z):
    Bs, Ls, Ds = Z.shape
    to = jnp.asarray(_T_ODD)
    te2 = jnp.asarray(_T_E2)
    te4 = jnp.asarray(_T_E4)
    tob = jnp.asarray(_T_ODD.reshape(1024, 1024).astype(jnp.bfloat16))
    te2b = jnp.asarray(_T_E2.astype(jnp.bfloat16))
    te4b = jnp.asarray(_T_E4.astype(jnp.bfloat16))
    zp = Z.reshape(Bs, 4, Ls // 4, Ds)

    dt = 512
    nj = Ds // dt
    ntiles = Bs * nj

    def _zmap(s):
        t = jnp.minimum(s, ntiles - 1)
        return (t // nj, 0, 0, t % nj)

    def _omap(s):
        t = jnp.maximum(s - 1, 0)
        return (t // nj, 0, t % nj)

    return pl.pallas_call(
        _freq_attn_kernel,
        grid=(ntiles + 1,),
        in_specs=[
            pl.BlockSpec((2, 512, 1024), lambda s: (0, 0, 0)),
            pl.BlockSpec((512, 512), lambda s: (0, 0)),
            pl.BlockSpec((512, 512), lambda s: (0, 0)),
            pl.BlockSpec((1024, 1024), lambda s: (0, 0)),
            pl.BlockSpec((512, 512), lambda s: (0, 0)),
            pl.BlockSpec((512, 512), lambda s: (0, 0)),
            pl.BlockSpec((1, 4, Ls // 4, dt), _zmap),
        ],
        out_specs=pl.BlockSpec((1, Ls, dt), _omap),
        out_shape=jax.ShapeDtypeStruct((Bs, Ls, Ds), jnp.float32),
        scratch_shapes=[pltpu.VMEM((2, 2 * _NF, dt), jnp.float32)],
        compiler_params=pltpu.CompilerParams(
            dimension_semantics=("arbitrary",)),
    )(to, te2, te4, tob, te2b, te4b, zp)


# speedup vs baseline: 1.2691x; 1.2691x over previous
import numpy as np

import jax
import jax.numpy as jnp
from jax.experimental import pallas as pl
from jax.experimental.pallas import tpu as pltpu

_L = 2048
_NF = 1024
_K = 5


def _build_tables():
    n1 = np.arange(1024, dtype=np.float64)[:, None]
    j1 = np.arange(512, dtype=np.float64)[None, :]
    ang_o = (2.0 * np.pi / 2048.0) * (2.0 * j1 + 1.0) * n1
    t_odd = np.concatenate([np.cos(ang_o), np.sin(ang_o)], axis=1)

    n2 = np.arange(512, dtype=np.float64)[:, None]
    j2 = np.arange(256, dtype=np.float64)[None, :]
    ang_e2 = (2.0 * np.pi / 1024.0) * (2.0 * j2 + 1.0) * n2
    t_e2 = np.concatenate([np.cos(ang_e2), np.sin(ang_e2)], axis=1)
    ang_e4 = (2.0 * np.pi / 512.0) * (j2 + 1.0) * n2
    t_e4 = np.concatenate([np.cos(ang_e4), np.sin(ang_e4)], axis=1)

    return (t_odd.astype(np.float32).reshape(2, 512, 1024),
            t_e2.astype(np.float32),
            t_e4.astype(np.float32))


_T_ODD, _T_E2, _T_E4 = _build_tables()


def _hdot(a, b):
    return jax.lax.dot_general(
        a, b, (((0,), (0,)), ((), ())),
        preferred_element_type=jnp.float32,
        precision=jax.lax.Precision.HIGHEST)


def _bdot(a, b):
    return jax.lax.dot_general(
        a, b, (((1,), (0,)), ((), ())),
        preferred_element_type=jnp.float32)


def _freq_attn_kernel(to_ref, te2_ref, te4_ref, tob_ref, te2b_ref, te4b_ref,
                      z_ref, o_ref):
    zb = z_ref[0]
    q0, q1, q2, q3 = zb[0], zb[1], zb[2], zb[3]
    c0 = q0 - q2
    c1 = q1 - q3
    a0 = q0 + q2
    a1 = q1 + q3
    ap = a0 + a1
    cp = a0 - a1

    odd = _hdot(to_ref[0], c0) + _hdot(to_ref[1], c1)
    e2 = _hdot(te2_ref[...], cp)
    e4 = _hdot(te4_ref[...], ap)

    Re = jnp.concatenate([odd[:512], e2[:256], e4[:256]], axis=0)
    Im = jnp.concatenate([odd[512:], e2[256:], e4[256:]], axis=0)
    amp = jnp.sqrt(Re * Re + Im * Im)
    r = jax.lax.broadcasted_iota(jnp.int32, amp.shape, 0)
    fidx = jnp.where(r < 512, 2 * r + 1,
                     jnp.where(r < 768, 4 * r - 2046, 4 * r - 3068))

    def body(_, w):
        m = jnp.max(w, axis=0, keepdims=True)
        idx = jnp.min(jnp.where(w == m, fidx, 2 * _NF), axis=0, keepdims=True)
        return jnp.where(fidx == idx, -1.0, w)

    work = jax.lax.fori_loop(0, _K, body, amp)
    sel = work < 0.0

    spec_o = jnp.where(jnp.concatenate([sel[:512], sel[:512]], axis=0),
                       odd, 0.0).astype(jnp.bfloat16)
    spec_e2 = jnp.where(jnp.concatenate([sel[512:768], sel[512:768]], axis=0),
                        e2, 0.0).astype(jnp.bfloat16)
    spec_e4 = jnp.where(jnp.concatenate([sel[768:], sel[768:]], axis=0),
                        e4, 0.0).astype(jnp.bfloat16)

    so = _bdot(tob_ref[...], spec_o)
    se2 = _bdot(te2b_ref[...], spec_e2)
    se4 = _bdot(te4b_ref[...], spec_e4)
    se = jnp.concatenate([se4 + se2, se4 - se2], axis=0)
    o_ref[0] = jnp.concatenate([se + so, se - so], axis=0)


def kernel(Z):
    Bs, Ls, Ds = Z.shape
    to = jnp.asarray(_T_ODD)
    te2 = jnp.asarray(_T_E2)
    te4 = jnp.asarray(_T_E4)
    tob = jnp.asarray(_T_ODD.reshape(1024, 1024).astype(jnp.bfloat16))
    te2b = jnp.asarray(_T_E2.astype(jnp.bfloat16))
    te4b = jnp.asarray(_T_E4.astype(jnp.bfloat16))
    zp = Z.reshape(Bs, 4, Ls // 4, Ds)

    dt = 512
    return pl.pallas_call(
        _freq_attn_kernel,
        grid=(Bs, Ds // dt),
        in_specs=[
            pl.BlockSpec((2, 512, 1024), lambda b, j: (0, 0, 0)),
            pl.BlockSpec((512, 512), lambda b, j: (0, 0)),
            pl.BlockSpec((512, 512), lambda b, j: (0, 0)),
            pl.BlockSpec((1024, 1024), lambda b, j: (0, 0)),
            pl.BlockSpec((512, 512), lambda b, j: (0, 0)),
            pl.BlockSpec((512, 512), lambda b, j: (0, 0)),
            pl.BlockSpec((1, 4, Ls // 4, dt), lambda b, j: (b, 0, 0, j)),
        ],
        out_specs=pl.BlockSpec((1, Ls, dt), lambda b, j: (b, 0, j)),
        out_shape=jax.ShapeDtypeStruct((Bs, Ls, Ds), jnp.float32),
        compiler_params=pltpu.CompilerParams(
            dimension_semantics=("parallel", "parallel")),
    )(to, te2, te4, tob, te2b, te4b, zp)
